# baseline (speedup 1.0000x reference)
import jax
import jax.numpy as jnp
from jax.experimental import pallas as pl
from jax.experimental.pallas import tpu as pltpu
from functools import partial

_N = 50000
_DYN = 128
_STATIC = 128
_HID = 256
_NLAYERS = 3
_NGRAPHS = 256
_R = 1000
_B = 256
_C = 256
_NT = (_N + _B - 1) // _B
_KMAX = 200000 // _C + _NT


def _mlp_body(with_agg, h_ref, st_ref, aggh_ref, aggs_ref, mask_ref, eps_ref,
              w1a_ref, w1b_ref, w2_ref, w3_ref, b1_ref, b2_ref, b3_ref, out_ref):
    h = h_ref[...]
    st = st_ref[...]
    if with_agg:
        e = eps_ref[0, 0]
        za = e * h + aggh_ref[...]
        zb = e * st + aggs_ref[...]
    else:
        za = h
        zb = st
    f32 = jnp.float32
    h1 = jnp.maximum(
        jax.lax.dot(za, w1a_ref[...], preferred_element_type=f32)
        + jax.lax.dot(zb, w1b_ref[...], preferred_element_type=f32)
        + b1_ref[0:1, :], 0.0)
    h2 = jnp.maximum(jax.lax.dot(h1, w2_ref[...], preferred_element_type=f32)
                     + b2_ref[0:1, :], 0.0)
    h3 = jnp.maximum(jax.lax.dot(h2, w3_ref[...], preferred_element_type=f32)
                     + b3_ref[0:1, :], 0.0)
    m = mask_ref[...]
    out_ref[...] = m * h3 + (1.0 - m) * h


def _run_mlp(h, st, aggh, aggs, mask, eps1, p, with_agg):
    grid = _N // _R
    row_spec = lambda w: pl.BlockSpec((_R, w), lambda i: (i, 0))
    full_spec = lambda a, b: pl.BlockSpec((a, b), lambda i: (0, 0))
    w1 = p['W1']
    in_specs = [
        row_spec(_DYN), row_spec(_STATIC), row_spec(_DYN),
        row_spec(_STATIC), row_spec(_DYN), full_spec(8, 128),
        full_spec(_DYN, _HID), full_spec(_STATIC, _HID),
        full_spec(_HID, _HID), full_spec(_HID, _DYN),
        full_spec(8, _HID), full_spec(8, _HID), full_spec(8, _DYN),
    ]
    eps_arr = jnp.broadcast_to(eps1.reshape(1, 1), (8, 128)).astype(jnp.float32)
    b1 = jnp.broadcast_to(p['b1'][None, :], (8, _HID))
    b2 = jnp.broadcast_to(p['b2'][None, :], (8, _HID))
    b3 = jnp.broadcast_to(p['b3'][None, :], (8, _DYN))
    return pl.pallas_call(
        partial(_mlp_body, with_agg),
        grid=(grid,),
        in_specs=in_specs,
        out_specs=pl.BlockSpec((_R, _DYN), lambda i: (i, 0)),
        out_shape=jax.ShapeDtypeStruct((_N, _DYN), jnp.float32),
    )(h, st, aggh, aggs, mask, eps_arr,
      w1[:_DYN], w1[_DYN:], p['W2'], p['W3'], b1, b2, b3)


def _scatter_body(ct_ref, mh_ref, ms_ref, dr_ref, aggh_ref, aggs_ref):
    k = pl.program_id(0)
    t = ct_ref[k]
    tp = ct_ref[jnp.maximum(k - 1, 0)]
    first = jnp.logical_or(k == 0, t != tp)

    @pl.when(first)
    def _():
        aggh_ref[...] = jnp.zeros_like(aggh_ref)
        aggs_ref[...] = jnp.zeros_like(aggs_ref)

    d = dr_ref[0]
    iota = jax.lax.broadcasted_iota(jnp.int32, (_B, _C), 0)
    oh = (iota == d).astype(jnp.float32)
    hp = jax.lax.Precision.HIGHEST
    aggh_ref[...] += jax.lax.dot(oh, mh_ref[...], precision=hp,
                                 preferred_element_type=jnp.float32)
    aggs_ref[...] += jax.lax.dot(oh, ms_ref[...], precision=hp,
                                 preferred_element_type=jnp.float32)


def _edge_plan(dst_sorted):
    e = dst_sorted.shape[0]
    starts = jnp.searchsorted(dst_sorted, jnp.arange(_NT + 1, dtype=jnp.int32) * _B)
    starts = starts.astype(jnp.int32)
    counts = starts[1:] - starts[:-1]
    nchunks = jnp.maximum((counts + _C - 1) // _C, 1)
    csum = jnp.cumsum(nchunks)
    offsets = csum - nchunks
    ks = jnp.arange(_KMAX, dtype=jnp.int32)
    chunk_tile = jnp.minimum(
        jnp.searchsorted(csum, ks, side='right').astype(jnp.int32), _NT - 1)
    local = ks - offsets[chunk_tile]
    estart = starts[chunk_tile] + local * _C
    eidx = estart[:, None] + jnp.arange(_C, dtype=jnp.int32)[None, :]
    valid = (eidx < starts[chunk_tile + 1][:, None]) & (eidx >= starts[chunk_tile][:, None])
    eidx_c = jnp.clip(eidx, 0, e - 1)
    dst_rel = jnp.where(valid, dst_sorted[eidx_c] - chunk_tile[:, None] * _B,
                        jnp.int32(-1))
    return chunk_tile, eidx_c.reshape(-1), valid.reshape(-1), dst_rel.reshape(_KMAX, 1, _C)


def _run_scatter(h, st, src_sel, em_flat, chunk_tile, dst_rel):
    mh = jnp.take(h, src_sel, axis=0) * em_flat[:, None]
    ms = jnp.take(st, src_sel, axis=0) * em_flat[:, None]
    grid_spec = pltpu.PrefetchScalarGridSpec(
        num_scalar_prefetch=1,
        grid=(_KMAX,),
        in_specs=[
            pl.BlockSpec((_C, _DYN), lambda k, ct: (k, 0)),
            pl.BlockSpec((_C, _STATIC), lambda k, ct: (k, 0)),
            pl.BlockSpec((1, 1, _C), lambda k, ct: (k, 0, 0)),
        ],
        out_specs=[
            pl.BlockSpec((_B, _DYN), lambda k, ct: (ct[k], 0)),
            pl.BlockSpec((_B, _STATIC), lambda k, ct: (ct[k], 0)),
        ],
    )
    aggh, aggs = pl.pallas_call(
        _scatter_body,
        grid_spec=grid_spec,
        out_shape=[jax.ShapeDtypeStruct((_NT * _B, _DYN), jnp.float32),
                   jax.ShapeDtypeStruct((_NT * _B, _STATIC), jnp.float32)],
    )(chunk_tile, mh, ms, dst_rel)
    return aggh[:_N], aggs[:_N]


def _pool_body(h_ref, bid_ref, wl_ref, bl_ref, out_ref, acc_ref):
    j = pl.program_id(0)
    bid = bid_ref[:, 0:1]
    iota = jax.lax.broadcasted_iota(jnp.int32, (_R, _NGRAPHS), 1)
    oh = (iota == bid).astype(jnp.float32)
    part = jax.lax.dot_general(oh, h_ref[...], (((0,), (0,)), ((), ())),
                               preferred_element_type=jnp.float32)

    @pl.when(j == 0)
    def _():
        acc_ref[...] = jnp.zeros_like(acc_ref)

    acc_ref[...] += part

    @pl.when(j == pl.num_programs(0) - 1)
    def _():
        out_ref[...] = jnp.maximum(
            jax.lax.dot(acc_ref[...], wl_ref[...],
                        preferred_element_type=jnp.float32) + bl_ref[0:1, :], 0.0)


def _run_pool(h, batch_ids, wlin, blin):
    bid2d = jnp.broadcast_to(batch_ids[:, None], (_N, 128))
    wl = jnp.zeros((_DYN, 128), jnp.float32).at[:, 0].set(wlin[:, 0])
    bl = jnp.zeros((8, 128), jnp.float32).at[:, 0].set(blin[0])
    out = pl.pallas_call(
        _pool_body,
        grid=(_N // _R,),
        in_specs=[
            pl.BlockSpec((_R, _DYN), lambda i: (i, 0)),
            pl.BlockSpec((_R, 128), lambda i: (i, 0)),
            pl.BlockSpec((_DYN, 128), lambda i: (0, 0)),
            pl.BlockSpec((8, 128), lambda i: (0, 0)),
        ],
        out_specs=pl.BlockSpec((_NGRAPHS, 128), lambda i: (0, 0)),
        out_shape=jax.ShapeDtypeStruct((_NGRAPHS, 128), jnp.float32),
        scratch_shapes=[pltpu.VMEM((_NGRAPHS, 128), jnp.float32)],
    )(h, bid2d, wl, bl)
    return out[:, :1]


def kernel(x, feature_mtx_static, edges_inner, edges_forward, edges_backward,
           layers, batch_ids, params):
    n = x.shape[0]
    st = feature_mtx_static
    h = jnp.concatenate([x, jnp.zeros((n, _DYN - 1), x.dtype)], axis=1)

    lmask = [jnp.broadcast_to((layers == l)[:, None], (n, _DYN)).astype(jnp.float32)
             for l in range(_NLAYERS)]
    e_in = params['eps_in'] + 1.0
    e_fw = params['eps_fw'] + 1.0
    e_bw = params['eps_bw'] + 1.0

    def prep(edges):
        src, dst = edges[0], edges[1]
        order = jnp.argsort(dst)
        src_s, dst_s = src[order], dst[order]
        chunk_tile, eidx, valid, dst_rel = _edge_plan(dst_s)
        src_sel = src_s[eidx]
        lsrc_sel = layers[src_sel]
        ldst_sel = layers[dst_s[eidx]]
        return src_sel, lsrc_sel, ldst_sel, valid, chunk_tile, dst_rel

    plan_i = prep(edges_inner)
    plan_f = prep(edges_forward)
    plan_b = prep(edges_backward)

    def gin(h, plan, la, lb, eps1, p, mask):
        src_sel, lsrc, ldst, valid, chunk_tile, dst_rel = plan
        em = ((lsrc == la) & (ldst == lb) & valid).astype(jnp.float32)
        aggh, aggs = _run_scatter(h, st, src_sel, em, chunk_tile, dst_rel)
        return _run_mlp(h, st, aggh, aggs, mask, eps1, p, True)

    def node(h, mask):
        z = jnp.zeros((n, _DYN), jnp.float32)
        return _run_mlp(h, st, z, z, mask, e_in, params['node'], False)

    for il in range(_NLAYERS):
        h = gin(h, plan_i, il, il, e_in, params['inlayer'], lmask[il])
        if il == _NLAYERS - 1:
            continue
        h = gin(h, plan_f, il, il + 1, e_fw, params['fwd'], lmask[il + 1])
        h = node(h, lmask[il + 1])
    for il in range(_NLAYERS - 1, 0, -1):
        h = gin(h, plan_b, il, il - 1, e_bw, params['bwd'], lmask[il - 1])
        h = gin(h, plan_i, il - 1, il - 1, e_in, params['inlayer'], lmask[il - 1])
        h = node(h, lmask[il - 1])
    return _run_pool(h, batch_ids, params['Wlin'], params['blin'])


# speedup vs baseline: 1.0020x; 1.0020x over previous
import jax
import jax.numpy as jnp
from jax.experimental import pallas as pl
from jax.experimental.pallas import tpu as pltpu
from functools import partial

_N = 50000
_DYN = 128
_STATIC = 128
_HID = 256
_NLAYERS = 3
_NGRAPHS = 256
_R = 1000
_B = 256
_C = 256
_NT = (_N + _B - 1) // _B
_KMAX = 200000 // _C + _NT


def _mlp_body(with_agg, h_ref, st_ref, aggh_ref, aggs_ref, mask_ref, eps_ref,
              w1a_ref, w1b_ref, w2_ref, w3_ref, b1_ref, b2_ref, b3_ref, out_ref):
    h = h_ref[...]
    st = st_ref[...]
    if with_agg:
        e = eps_ref[0, 0]
        za = e * h + aggh_ref[...]
        zb = e * st + aggs_ref[...]
    else:
        za = h
        zb = st
    f32 = jnp.float32
    h1 = jnp.maximum(
        jax.lax.dot(za, w1a_ref[...], preferred_element_type=f32)
        + jax.lax.dot(zb, w1b_ref[...], preferred_element_type=f32)
        + b1_ref[0:1, :], 0.0)
    h2 = jnp.maximum(jax.lax.dot(h1, w2_ref[...], preferred_element_type=f32)
                     + b2_ref[0:1, :], 0.0)
    h3 = jnp.maximum(jax.lax.dot(h2, w3_ref[...], preferred_element_type=f32)
                     + b3_ref[0:1, :], 0.0)
    m = mask_ref[...]
    out_ref[...] = m * h3 + (1.0 - m) * h


def _run_mlp(h, st, aggh, aggs, mask, eps1, p, with_agg):
    grid = _N // _R
    row_spec = lambda w: pl.BlockSpec((_R, w), lambda i: (i, 0))
    full_spec = lambda a, b: pl.BlockSpec((a, b), lambda i: (0, 0))
    w1 = p['W1']
    in_specs = [
        row_spec(_DYN), row_spec(_STATIC), row_spec(_DYN),
        row_spec(_STATIC), row_spec(_DYN), full_spec(8, 128),
        full_spec(_DYN, _HID), full_spec(_STATIC, _HID),
        full_spec(_HID, _HID), full_spec(_HID, _DYN),
        full_spec(8, _HID), full_spec(8, _HID), full_spec(8, _DYN),
    ]
    eps_arr = jnp.broadcast_to(eps1.reshape(1, 1), (8, 128)).astype(jnp.float32)
    b1 = jnp.broadcast_to(p['b1'][None, :], (8, _HID))
    b2 = jnp.broadcast_to(p['b2'][None, :], (8, _HID))
    b3 = jnp.broadcast_to(p['b3'][None, :], (8, _DYN))
    return pl.pallas_call(
        partial(_mlp_body, with_agg),
        grid=(grid,),
        in_specs=in_specs,
        out_specs=pl.BlockSpec((_R, _DYN), lambda i: (i, 0)),
        out_shape=jax.ShapeDtypeStruct((_N, _DYN), jnp.float32),
    )(h, st, aggh, aggs, mask, eps_arr,
      w1[:_DYN], w1[_DYN:], p['W2'], p['W3'], b1, b2, b3)


def _mlp_body_noagg(h_ref, st_ref, mask_ref,
                    w1a_ref, w1b_ref, w2_ref, w3_ref, b1_ref, b2_ref, b3_ref,
                    out_ref):
    h = h_ref[...]
    st = st_ref[...]
    f32 = jnp.float32
    h1 = jnp.maximum(
        jax.lax.dot(h, w1a_ref[...], preferred_element_type=f32)
        + jax.lax.dot(st, w1b_ref[...], preferred_element_type=f32)
        + b1_ref[0:1, :], 0.0)
    h2 = jnp.maximum(jax.lax.dot(h1, w2_ref[...], preferred_element_type=f32)
                     + b2_ref[0:1, :], 0.0)
    h3 = jnp.maximum(jax.lax.dot(h2, w3_ref[...], preferred_element_type=f32)
                     + b3_ref[0:1, :], 0.0)
    m = mask_ref[...]
    out_ref[...] = m * h3 + (1.0 - m) * h


def _run_mlp_noagg(h, st, mask, p):
    grid = _N // _R
    row_spec = lambda w: pl.BlockSpec((_R, w), lambda i: (i, 0))
    full_spec = lambda a, b: pl.BlockSpec((a, b), lambda i: (0, 0))
    w1 = p['W1']
    in_specs = [
        row_spec(_DYN), row_spec(_STATIC), row_spec(_DYN),
        full_spec(_DYN, _HID), full_spec(_STATIC, _HID),
        full_spec(_HID, _HID), full_spec(_HID, _DYN),
        full_spec(8, _HID), full_spec(8, _HID), full_spec(8, _DYN),
    ]
    b1 = jnp.broadcast_to(p['b1'][None, :], (8, _HID))
    b2 = jnp.broadcast_to(p['b2'][None, :], (8, _HID))
    b3 = jnp.broadcast_to(p['b3'][None, :], (8, _DYN))
    return pl.pallas_call(
        _mlp_body_noagg,
        grid=(grid,),
        in_specs=in_specs,
        out_specs=pl.BlockSpec((_R, _DYN), lambda i: (i, 0)),
        out_shape=jax.ShapeDtypeStruct((_N, _DYN), jnp.float32),
    )(h, st, mask, w1[:_DYN], w1[_DYN:], p['W2'], p['W3'], b1, b2, b3)


def _scatter_body(ct_ref, mh_ref, ms_ref, dr_ref, aggh_ref, aggs_ref):
    k = pl.program_id(0)
    t = ct_ref[k]
    tp = ct_ref[jnp.maximum(k - 1, 0)]
    first = jnp.logical_or(k == 0, t != tp)

    @pl.when(first)
    def _():
        aggh_ref[...] = jnp.zeros_like(aggh_ref)
        aggs_ref[...] = jnp.zeros_like(aggs_ref)

    d = dr_ref[0]
    iota = jax.lax.broadcasted_iota(jnp.int32, (_B, _C), 0)
    oh = (iota == d).astype(jnp.float32)
    hp = jax.lax.Precision.HIGHEST
    aggh_ref[...] += jax.lax.dot(oh, mh_ref[...], precision=hp,
                                 preferred_element_type=jnp.float32)
    aggs_ref[...] += jax.lax.dot(oh, ms_ref[...], precision=hp,
                                 preferred_element_type=jnp.float32)


def _edge_plan(dst_sorted):
    e = dst_sorted.shape[0]
    starts = jnp.searchsorted(dst_sorted, jnp.arange(_NT + 1, dtype=jnp.int32) * _B)
    starts = starts.astype(jnp.int32)
    counts = starts[1:] - starts[:-1]
    nchunks = jnp.maximum((counts + _C - 1) // _C, 1)
    csum = jnp.cumsum(nchunks)
    offsets = csum - nchunks
    ks = jnp.arange(_KMAX, dtype=jnp.int32)
    chunk_tile = jnp.minimum(
        jnp.searchsorted(csum, ks, side='right').astype(jnp.int32), _NT - 1)
    local = ks - offsets[chunk_tile]
    estart = starts[chunk_tile] + local * _C
    eidx = estart[:, None] + jnp.arange(_C, dtype=jnp.int32)[None, :]
    valid = (eidx < starts[chunk_tile + 1][:, None]) & (eidx >= starts[chunk_tile][:, None])
    eidx_c = jnp.clip(eidx, 0, e - 1)
    dst_rel = jnp.where(valid, dst_sorted[eidx_c] - chunk_tile[:, None] * _B,
                        jnp.int32(-1))
    return chunk_tile, eidx_c.reshape(-1), valid.reshape(-1), dst_rel.reshape(_KMAX, 1, _C)


def _run_scatter(h, st, src_sel, em_flat, chunk_tile, dst_rel):
    mh = jnp.take(h, src_sel, axis=0) * em_flat[:, None]
    ms = jnp.take(st, src_sel, axis=0) * em_flat[:, None]
    grid_spec = pltpu.PrefetchScalarGridSpec(
        num_scalar_prefetch=1,
        grid=(_KMAX,),
        in_specs=[
            pl.BlockSpec((_C, _DYN), lambda k, ct: (k, 0)),
            pl.BlockSpec((_C, _STATIC), lambda k, ct: (k, 0)),
            pl.BlockSpec((1, 1, _C), lambda k, ct: (k, 0, 0)),
        ],
        out_specs=[
            pl.BlockSpec((_B, _DYN), lambda k, ct: (ct[k], 0)),
            pl.BlockSpec((_B, _STATIC), lambda k, ct: (ct[k], 0)),
        ],
    )
    aggh, aggs = pl.pallas_call(
        _scatter_body,
        grid_spec=grid_spec,
        out_shape=[jax.ShapeDtypeStruct((_NT * _B, _DYN), jnp.float32),
                   jax.ShapeDtypeStruct((_NT * _B, _STATIC), jnp.float32)],
    )(chunk_tile, mh, ms, dst_rel)
    return aggh[:_N], aggs[:_N]


def _pool_body(h_ref, bid_ref, wl_ref, bl_ref, out_ref, acc_ref):
    j = pl.program_id(0)
    bid = bid_ref[:, 0:1]
    iota = jax.lax.broadcasted_iota(jnp.int32, (_R, _NGRAPHS), 1)
    oh = (iota == bid).astype(jnp.float32)
    part = jax.lax.dot_general(oh, h_ref[...], (((0,), (0,)), ((), ())),
                               preferred_element_type=jnp.float32)

    @pl.when(j == 0)
    def _():
        acc_ref[...] = jnp.zeros_like(acc_ref)

    acc_ref[...] += part

    @pl.when(j == pl.num_programs(0) - 1)
    def _():
        out_ref[...] = jnp.maximum(
            jax.lax.dot(acc_ref[...], wl_ref[...],
                        preferred_element_type=jnp.float32) + bl_ref[0:1, :], 0.0)


def _run_pool(h, batch_ids, wlin, blin):
    bid2d = jnp.broadcast_to(batch_ids[:, None], (_N, 128))
    wl = jnp.zeros((_DYN, 128), jnp.float32).at[:, 0].set(wlin[:, 0])
    bl = jnp.zeros((8, 128), jnp.float32).at[:, 0].set(blin[0])
    out = pl.pallas_call(
        _pool_body,
        grid=(_N // _R,),
        in_specs=[
            pl.BlockSpec((_R, _DYN), lambda i: (i, 0)),
            pl.BlockSpec((_R, 128), lambda i: (i, 0)),
            pl.BlockSpec((_DYN, 128), lambda i: (0, 0)),
            pl.BlockSpec((8, 128), lambda i: (0, 0)),
        ],
        out_specs=pl.BlockSpec((_NGRAPHS, 128), lambda i: (0, 0)),
        out_shape=jax.ShapeDtypeStruct((_NGRAPHS, 128), jnp.float32),
        scratch_shapes=[pltpu.VMEM((_NGRAPHS, 128), jnp.float32)],
    )(h, bid2d, wl, bl)
    return out[:, :1]


def kernel(x, feature_mtx_static, edges_inner, edges_forward, edges_backward,
           layers, batch_ids, params):
    n = x.shape[0]
    st = feature_mtx_static
    h = jnp.concatenate([x, jnp.zeros((n, _DYN - 1), x.dtype)], axis=1)

    lmask = [jnp.broadcast_to((layers == l)[:, None], (n, _DYN)).astype(jnp.float32)
             for l in range(_NLAYERS)]
    e_in = params['eps_in'] + 1.0
    e_fw = params['eps_fw'] + 1.0
    e_bw = params['eps_bw'] + 1.0

    def prep(edges):
        src, dst = edges[0], edges[1]
        order = jnp.argsort(dst)
        src_s, dst_s = src[order], dst[order]
        chunk_tile, eidx, valid, dst_rel = _edge_plan(dst_s)
        src_sel = src_s[eidx]
        lsrc_sel = layers[src_sel]
        ldst_sel = layers[dst_s[eidx]]
        return src_sel, lsrc_sel, ldst_sel, valid, chunk_tile, dst_rel

    plan_i = prep(edges_inner)
    plan_f = prep(edges_forward)
    plan_b = prep(edges_backward)

    def gin(h, plan, la, lb, eps1, p, mask):
        src_sel, lsrc, ldst, valid, chunk_tile, dst_rel = plan
        em = ((lsrc == la) & (ldst == lb) & valid).astype(jnp.float32)
        aggh, aggs = _run_scatter(h, st, src_sel, em, chunk_tile, dst_rel)
        return _run_mlp(h, st, aggh, aggs, mask, eps1, p, True)

    def node(h, mask):
        return _run_mlp_noagg(h, st, mask, params['node'])

    for il in range(_NLAYERS):
        h = gin(h, plan_i, il, il, e_in, params['inlayer'], lmask[il])
        if il == _NLAYERS - 1:
            continue
        h = gin(h, plan_f, il, il + 1, e_fw, params['fwd'], lmask[il + 1])
        h = node(h, lmask[il + 1])
    for il in range(_NLAYERS - 1, 0, -1):
        h = gin(h, plan_b, il, il - 1, e_bw, params['bwd'], lmask[il - 1])
        h = gin(h, plan_i, il - 1, il - 1, e_in, params['inlayer'], lmask[il - 1])
        h = node(h, lmask[il - 1])
    return _run_pool(h, batch_ids, params['Wlin'], params['blin'])
